# baseline (speedup 1.0000x reference)
import functools

import jax
import jax.numpy as jnp
import numpy as np
from jax import lax
from jax.experimental import pallas as pl
from jax.experimental.pallas import tpu as pltpu
from jax.experimental.pallas import tpu_sc as plsc

F32 = jnp.float32

HIDDEN = 128
H2 = 256
N_GRID = 32768
N_QUERY = 10000
N_EDGES = 320000
NQ_PAD = 10240

NUM_CORES = 2
NUM_SUBCORES = 16
NUM_WORKERS = NUM_CORES * NUM_SUBCORES
EDGES_PER_WORKER = N_EDGES // NUM_WORKERS
CHUNK = 80
NUM_CHUNKS = EDGES_PER_WORKER // CHUNK
ROWS_PER_TILE = NQ_PAD // NUM_SUBCORES

MAX_WAVELENGTH = 10000.0


def _gelu(v):
    return 0.5 * v * (1.0 + lax.erf(v * np.float32(0.7071067811865476)))


def _pack_bf16_pair(a):
    lo = lax.bitcast_convert_type(a[:, :HIDDEN].astype(jnp.bfloat16), jnp.uint16)
    hi = lax.bitcast_convert_type(a[:, HIDDEN:].astype(jnp.bfloat16), jnp.uint16)
    return (hi.astype(jnp.uint32) << 16) | lo.astype(jnp.uint32)


def _a_body(x_ref, wp_ref, bp_ref, w1t_ref, a_ref):
    xf = jnp.dot(x_ref[...], wp_ref[...], preferred_element_type=F32) + bp_ref[...]
    a_ref[...] = _pack_bf16_pair(jnp.dot(xf, w1t_ref[...], preferred_element_type=F32))


def _make_a(xf2, w_proj, b_proj, w1t):
    blk = 2048
    return pl.pallas_call(
        _a_body,
        grid=(N_GRID // blk,),
        in_specs=[
            pl.BlockSpec((blk, 3), lambda i: (i, 0)),
            pl.BlockSpec((3, HIDDEN), lambda i: (0, 0)),
            pl.BlockSpec((1, HIDDEN), lambda i: (0, 0)),
            pl.BlockSpec((HIDDEN, H2), lambda i: (0, 0)),
        ],
        out_specs=pl.BlockSpec((blk, HIDDEN), lambda i: (i, 0)),
        out_shape=jax.ShapeDtypeStruct((N_GRID, HIDDEN), jnp.uint32),
    )(xf2, w_proj, b_proj, w1t)


_N_FREQ = 21
_EFF = 42


def _b_body(q_ref, w1b_ref, bm1_ref, om_ref, b_ref):
    omega = om_ref[...]
    pieces = []
    for d in range(3):
        prod = q_ref[:, d : d + 1] * omega
        pieces.append(jnp.sin(prod))
        pieces.append(jnp.cos(prod))
    pieces.append(jnp.zeros((q_ref.shape[0], 2), F32))
    emb = jnp.concatenate(pieces, axis=1)
    b_ref[...] = _pack_bf16_pair(
        jnp.dot(emb, w1b_ref[...], preferred_element_type=F32) + bm1_ref[...])


def _make_b(query_pos, w1b, b_m1):
    blk = 1000
    omega = jnp.asarray(
        (1.0 / (MAX_WAVELENGTH ** (np.arange(0, _EFF, 2, dtype=np.float32) / _EFF)))
        .reshape(1, _N_FREQ))
    return pl.pallas_call(
        _b_body,
        grid=(N_QUERY // blk,),
        in_specs=[
            pl.BlockSpec((blk, 3), lambda i: (i, 0)),
            pl.BlockSpec((HIDDEN, H2), lambda i: (0, 0)),
            pl.BlockSpec((1, H2), lambda i: (0, 0)),
            pl.BlockSpec((1, _N_FREQ), lambda i: (0, 0)),
        ],
        out_specs=pl.BlockSpec((blk, HIDDEN), lambda i: (i, 0)),
        out_shape=jax.ShapeDtypeStruct((N_QUERY, HIDDEN), jnp.uint32),
    )(query_pos, w1b, b_m1, omega)


def _sc_gather_body(a_hbm, b_hbm, gi_hbm, qi_hbm, ha_hbm, hb_hbm,
                    gi_all, qi_all, ar0, br0, ar1, br1, sg0, sg1, so0, so1):
    c = lax.axis_index("c")
    s = lax.axis_index("s")
    wid = s * NUM_CORES + c
    base = wid * EDGES_PER_WORKER

    pltpu.sync_copy(gi_hbm.at[pl.ds(base, EDGES_PER_WORKER)], gi_all)
    pltpu.sync_copy(qi_hbm.at[pl.ds(base, EDGES_PER_WORKER)], qi_all)

    def g_refs(ch, ar, br):
        gi = gi_all.at[pl.ds(ch * CHUNK, CHUNK)]
        qi = qi_all.at[pl.ds(ch * CHUNK, CHUNK)]
        return (a_hbm.at[gi], ar), (b_hbm.at[qi], br)

    def start_gather(ch, ar, br, sg):
        (sa, da), (sb, db) = g_refs(ch, ar, br)
        pltpu.async_copy(sa, da, sg)
        pltpu.async_copy(sb, db, sg)

    def wait_gather(ch, ar, br, sg):
        (sa, da), (sb, db) = g_refs(ch, ar, br)
        pltpu.make_async_copy(sa, da, sg).wait()
        pltpu.make_async_copy(sb, db, sg).wait()

    def start_out(ch, ar, br, so):
        off = base + ch * CHUNK
        pltpu.async_copy(ar, ha_hbm.at[pl.ds(off, CHUNK)], so)
        pltpu.async_copy(br, hb_hbm.at[pl.ds(off, CHUNK)], so)

    def wait_out(ch, ar, br, so):
        off = base + ch * CHUNK
        pltpu.make_async_copy(ar, ha_hbm.at[pl.ds(off, CHUNK)], so).wait()
        pltpu.make_async_copy(br, hb_hbm.at[pl.ds(off, CHUNK)], so).wait()

    start_gather(0, ar0, br0, sg0)

    def body(j, _):
        c0 = 2 * j
        c1 = 2 * j + 1

        @pl.when(j > 0)
        def _():
            wait_out(c1 - 2, ar1, br1, so1)

        start_gather(c1, ar1, br1, sg1)
        wait_gather(c0, ar0, br0, sg0)
        start_out(c0, ar0, br0, so0)

        @pl.when(c0 + 2 < NUM_CHUNKS)
        def _():
            wait_out(c0, ar0, br0, so0)
            start_gather(c0 + 2, ar0, br0, sg0)

        wait_gather(c1, ar1, br1, sg1)
        start_out(c1, ar1, br1, so1)
        return 0

    lax.fori_loop(0, NUM_CHUNKS // 2, body, 0, unroll=False)
    last = NUM_CHUNKS - 1
    wait_gather(last, ar0, br0, sg0)
    start_out(last, ar0, br0, so0)
    wait_out(last, ar0, br0, so0)
    wait_out(NUM_CHUNKS - 2, ar1, br1, so1)


def _sc_gather(a, b, g_idx, q_idx):
    mesh = plsc.VectorSubcoreMesh(core_axis_name="c", subcore_axis_name="s")
    return pl.kernel(
        _sc_gather_body,
        out_type=(
            jax.ShapeDtypeStruct((N_EDGES, HIDDEN), jnp.uint32),
            jax.ShapeDtypeStruct((N_EDGES, HIDDEN), jnp.uint32),
        ),
        mesh=mesh,
        scratch_types=[
            pltpu.VMEM((EDGES_PER_WORKER,), jnp.int32),
            pltpu.VMEM((EDGES_PER_WORKER,), jnp.int32),
            pltpu.VMEM((CHUNK, HIDDEN), jnp.uint32),
            pltpu.VMEM((CHUNK, HIDDEN), jnp.uint32),
            pltpu.VMEM((CHUNK, HIDDEN), jnp.uint32),
            pltpu.VMEM((CHUNK, HIDDEN), jnp.uint32),
            pltpu.SemaphoreType.DMA,
            pltpu.SemaphoreType.DMA,
            pltpu.SemaphoreType.DMA,
            pltpu.SemaphoreType.DMA,
        ],
    )(a, b, g_idx, q_idx)


def _unpack_pair(p):
    lo = lax.bitcast_convert_type((p & 0xFFFF).astype(jnp.uint16), jnp.bfloat16)
    hi = lax.bitcast_convert_type((p >> 16).astype(jnp.uint16), jnp.bfloat16)
    return lo.astype(F32), hi.astype(F32)


def _mlp_body(ha_ref, hb_ref, w2_ref, b2_ref, v_ref):
    alo, ahi = _unpack_pair(ha_ref[...])
    blo, bhi = _unpack_pair(hb_ref[...])
    h = jnp.concatenate([alo + blo, ahi + bhi], axis=1)
    u = _gelu(h).astype(jnp.bfloat16)
    w2 = w2_ref[...].astype(jnp.bfloat16)
    v_ref[...] = _gelu(jnp.dot(u, w2, preferred_element_type=F32) + b2_ref[...])


def _make_v(ha, hb, w_m2, b_m2):
    blk = 3200
    return pl.pallas_call(
        _mlp_body,
        grid=(N_EDGES // blk,),
        in_specs=[
            pl.BlockSpec((blk, HIDDEN), lambda i: (i, 0)),
            pl.BlockSpec((blk, HIDDEN), lambda i: (i, 0)),
            pl.BlockSpec((H2, HIDDEN), lambda i: (0, 0)),
            pl.BlockSpec((1, HIDDEN), lambda i: (0, 0)),
        ],
        out_specs=pl.BlockSpec((blk, HIDDEN), lambda i: (i, 0)),
        out_shape=jax.ShapeDtypeStruct((N_EDGES, HIDDEN), F32),
    )(ha, hb, w_m2, b_m2)


NQ_HALF = NQ_PAD // 2
ACC_ROWS = 5248
ROWS_PER_TILE_S = ACC_ROWS // NUM_SUBCORES
DUMP_ROW = NQ_HALF
EDGES_PER_SUBCORE = N_EDGES // NUM_SUBCORES
NUM_CHUNKS_S = EDGES_PER_SUBCORE // CHUNK


def _fill_const(buf, rows, val16):
    def frow(r, _):
        for j in range(HIDDEN // 16):
            buf[r, pl.ds(j * 16, 16)] = val16
        return 0

    lax.fori_loop(0, rows, frow, 0, unroll=False)


def _zero_acc_slice(acc_s, v_v, r0):
    for k in range(4):
        pltpu.sync_copy(v_v, acc_s.at[pl.ds(r0 + k * CHUNK, CHUNK)])
    pltpu.sync_copy(v_v.at[pl.ds(0, 8)], acc_s.at[pl.ds(r0 + 4 * CHUNK, 8)])


def _sc_scatter_body(v_hbm, qi_hbm, sums_hbm, cnts_hbm,
                     qi_all, qi2_all, v0, v1, acc_s, sv0, sv1, ss0, ss1, s2):
    c = lax.axis_index("c")
    s = lax.axis_index("s")
    base = s * EDGES_PER_SUBCORE
    lo = c * NQ_HALF
    r0 = s * ROWS_PER_TILE_S

    pltpu.sync_copy(qi_hbm.at[pl.ds(base, EDGES_PER_SUBCORE)], qi_all)

    def premap(i, _):
        for k in range(CHUNK // 16):
            q = qi_all[pl.ds(i * CHUNK + k * 16, 16)] - lo
            in_rng = (q >= 0) & (q < NQ_HALF)
            qi2_all[i, pl.ds(k * 16, 16)] = jnp.where(in_rng, q, DUMP_ROW)
        return 0

    lax.fori_loop(0, NUM_CHUNKS_S, premap, 0, unroll=False)

    _fill_const(v0, CHUNK, jnp.zeros((16,), F32))
    _zero_acc_slice(acc_s, v0, r0)
    plsc.subcore_barrier()

    def start_vload(ch, vb, sv):
        pltpu.async_copy(v_hbm.at[pl.ds(base + ch * CHUNK, CHUNK)], vb, sv)

    def wait_vload(ch, vb, sv):
        pltpu.make_async_copy(v_hbm.at[pl.ds(base + ch * CHUNK, CHUNK)], vb, sv).wait()

    def start_scatter(ch, vb, ss):
        pltpu.async_copy(vb, acc_s.at[qi2_all.at[ch]], ss, add=True)

    def wait_scatter(ch, vb, ss):
        pltpu.make_async_copy(vb, acc_s.at[qi2_all.at[ch]], ss).wait()

    start_vload(0, v0, sv0)

    def body(j, _):
        c0 = 2 * j
        c1 = 2 * j + 1

        @pl.when(j > 0)
        def _():
            wait_scatter(c1 - 2, v1, ss1)

        start_vload(c1, v1, sv1)
        wait_vload(c0, v0, sv0)
        start_scatter(c0, v0, ss0)

        @pl.when(c0 + 2 < NUM_CHUNKS_S)
        def _():
            wait_scatter(c0, v0, ss0)
            start_vload(c0 + 2, v0, sv0)

        wait_vload(c1, v1, sv1)
        start_scatter(c1, v1, ss1)
        return 0

    lax.fori_loop(0, NUM_CHUNKS_S // 2, body, 0, unroll=False)
    wait_scatter(NUM_CHUNKS_S - 2, v0, ss0)
    wait_scatter(NUM_CHUNKS_S - 1, v1, ss1)
    plsc.subcore_barrier()

    pltpu.sync_copy(acc_s.at[pl.ds(r0, ROWS_PER_TILE_S)],
                    sums_hbm.at[c, pl.ds(r0, ROWS_PER_TILE_S)])
    _fill_const(v0, CHUNK, jnp.zeros((16,), F32))
    _zero_acc_slice(acc_s, v0, r0)
    plsc.subcore_barrier()

    _fill_const(v0, CHUNK, jnp.ones((16,), F32))

    def body2(j, _):
        start_scatter(2 * j, v0, s2)
        start_scatter(2 * j + 1, v0, s2)

        @pl.when(j > 0)
        def _():
            wait_scatter(2 * j - 2, v0, s2)
            wait_scatter(2 * j - 1, v0, s2)

        return 0

    lax.fori_loop(0, NUM_CHUNKS_S // 2, body2, 0, unroll=False)
    wait_scatter(NUM_CHUNKS_S - 2, v0, s2)
    wait_scatter(NUM_CHUNKS_S - 1, v0, s2)
    plsc.subcore_barrier()

    pltpu.sync_copy(acc_s.at[pl.ds(r0, ROWS_PER_TILE_S)],
                    cnts_hbm.at[c, pl.ds(r0, ROWS_PER_TILE_S)])


def _sc_scatter(v, q_idx):
    mesh = plsc.VectorSubcoreMesh(core_axis_name="c", subcore_axis_name="s")
    return pl.kernel(
        _sc_scatter_body,
        out_type=(
            jax.ShapeDtypeStruct((NUM_CORES, ACC_ROWS, HIDDEN), F32),
            jax.ShapeDtypeStruct((NUM_CORES, ACC_ROWS, HIDDEN), F32),
        ),
        mesh=mesh,
        scratch_types=[
            pltpu.VMEM((EDGES_PER_SUBCORE,), jnp.int32),
            pltpu.VMEM((NUM_CHUNKS_S, CHUNK), jnp.int32),
            pltpu.VMEM((CHUNK, HIDDEN), F32),
            pltpu.VMEM((CHUNK, HIDDEN), F32),
            pltpu.VMEM_SHARED((ACC_ROWS, HIDDEN), F32),
            pltpu.SemaphoreType.DMA,
            pltpu.SemaphoreType.DMA,
            pltpu.SemaphoreType.DMA,
            pltpu.SemaphoreType.DMA,
            pltpu.SemaphoreType.DMA,
        ],
    )(v, q_idx)


def _fin_body(s_ref, cnt_ref, w3_ref, b3_ref,
              wp1_ref, bp1_ref, wp2_ref, bp2_ref, o_ref):
    sums = s_ref[...]
    cnt = cnt_ref[:, 0:1]
    mbar = sums / jnp.maximum(cnt, 1.0)
    m = jnp.dot(mbar, w3_ref[...], preferred_element_type=F32) + b3_ref[...]
    m = jnp.where(cnt > 0.0, m, 0.0)
    z = _gelu(jnp.dot(m, wp1_ref[...], preferred_element_type=F32) + bp1_ref[...])
    o_ref[...] = jnp.dot(z, wp2_ref[...], preferred_element_type=F32) + bp2_ref[...]


def _finalize(s, cnt, w_m3, b_m3, w_p1, b_p1, w_p2, b_p2):
    blk = 1024
    n_out = 4
    return pl.pallas_call(
        _fin_body,
        grid=(NQ_PAD // blk,),
        in_specs=[
            pl.BlockSpec((blk, HIDDEN), lambda i: (i, 0)),
            pl.BlockSpec((blk, HIDDEN), lambda i: (i, 0)),
            pl.BlockSpec((HIDDEN, HIDDEN), lambda i: (0, 0)),
            pl.BlockSpec((1, HIDDEN), lambda i: (0, 0)),
            pl.BlockSpec((HIDDEN, HIDDEN), lambda i: (0, 0)),
            pl.BlockSpec((1, HIDDEN), lambda i: (0, 0)),
            pl.BlockSpec((HIDDEN, n_out), lambda i: (0, 0)),
            pl.BlockSpec((1, n_out), lambda i: (0, 0)),
        ],
        out_specs=pl.BlockSpec((blk, n_out), lambda i: (i, 0)),
        out_shape=jax.ShapeDtypeStruct((NQ_PAD, n_out), F32),
    )(s, cnt, w_m3, b_m3, w_p1, b_p1, w_p2, b_p2)


def kernel(x, query_pos, grid_to_query_edges, W_proj, b_proj, W_m1, b_m1,
           W_m2, b_m2, W_m3, b_m3, W_p1, b_p1, W_p2, b_p2):
    xf2 = x.reshape(-1, x.shape[-1]).astype(F32)
    q_idx = grid_to_query_edges[:, 0].astype(jnp.int32)
    g_idx = grid_to_query_edges[:, 1].astype(jnp.int32)

    w1t = W_m1[:HIDDEN]
    w1b = W_m1[HIDDEN:]

    a = _make_a(xf2, W_proj, b_proj.reshape(1, -1), w1t)
    b = _make_b(query_pos, w1b, b_m1.reshape(1, -1))
    ha, hb = _sc_gather(a, b, g_idx, q_idx)
    v = _make_v(ha, hb, W_m2, b_m2.reshape(1, -1))
    sums_p, cnts_p = _sc_scatter(v, q_idx)
    sums = jnp.concatenate([sums_p[0, :NQ_HALF], sums_p[1, :NQ_HALF]], axis=0)
    cnts = jnp.concatenate([cnts_p[0, :NQ_HALF], cnts_p[1, :NQ_HALF]], axis=0)
    out = _finalize(sums, cnts,
                    W_m3, b_m3.reshape(1, -1), W_p1, b_p1.reshape(1, -1),
                    W_p2, b_p2.reshape(1, -1))
    return out[:N_QUERY]


# speedup vs baseline: 1.0024x; 1.0024x over previous
import functools

import jax
import jax.numpy as jnp
import numpy as np
from jax import lax
from jax.experimental import pallas as pl
from jax.experimental.pallas import tpu as pltpu
from jax.experimental.pallas import tpu_sc as plsc

F32 = jnp.float32

HIDDEN = 128
H2 = 256
N_GRID = 32768
N_QUERY = 10000
N_EDGES = 320000
NQ_PAD = 10240

NUM_CORES = 2
NUM_SUBCORES = 16
NUM_WORKERS = NUM_CORES * NUM_SUBCORES
EDGES_PER_WORKER = N_EDGES // NUM_WORKERS
CHUNK = 80
NUM_CHUNKS = EDGES_PER_WORKER // CHUNK
ROWS_PER_TILE = NQ_PAD // NUM_SUBCORES

MAX_WAVELENGTH = 10000.0


def _gelu(v):
    return 0.5 * v * (1.0 + lax.erf(v * np.float32(0.7071067811865476)))


def _pack_bf16_pair(a):
    lo = lax.bitcast_convert_type(a[:, :HIDDEN].astype(jnp.bfloat16), jnp.uint16)
    hi = lax.bitcast_convert_type(a[:, HIDDEN:].astype(jnp.bfloat16), jnp.uint16)
    return (hi.astype(jnp.uint32) << 16) | lo.astype(jnp.uint32)


def _a_body(x_ref, wp_ref, bp_ref, w1t_ref, a_ref):
    xf = jnp.dot(x_ref[...], wp_ref[...], preferred_element_type=F32) + bp_ref[...]
    a_ref[...] = _pack_bf16_pair(jnp.dot(xf, w1t_ref[...], preferred_element_type=F32))


def _make_a(xf2, w_proj, b_proj, w1t):
    blk = 2048
    return pl.pallas_call(
        _a_body,
        grid=(N_GRID // blk,),
        in_specs=[
            pl.BlockSpec((blk, 3), lambda i: (i, 0)),
            pl.BlockSpec((3, HIDDEN), lambda i: (0, 0)),
            pl.BlockSpec((1, HIDDEN), lambda i: (0, 0)),
            pl.BlockSpec((HIDDEN, H2), lambda i: (0, 0)),
        ],
        out_specs=pl.BlockSpec((blk, HIDDEN), lambda i: (i, 0)),
        out_shape=jax.ShapeDtypeStruct((N_GRID, HIDDEN), jnp.uint32),
    )(xf2, w_proj, b_proj, w1t)


_N_FREQ = 21
_EFF = 42


def _b_body(q_ref, w1b_ref, bm1_ref, om_ref, b_ref):
    omega = om_ref[...]
    pieces = []
    for d in range(3):
        prod = q_ref[:, d : d + 1] * omega
        pieces.append(jnp.sin(prod))
        pieces.append(jnp.cos(prod))
    pieces.append(jnp.zeros((q_ref.shape[0], 2), F32))
    emb = jnp.concatenate(pieces, axis=1)
    b_ref[...] = _pack_bf16_pair(
        jnp.dot(emb, w1b_ref[...], preferred_element_type=F32) + bm1_ref[...])


def _make_b(query_pos, w1b, b_m1):
    blk = 1000
    omega = jnp.asarray(
        (1.0 / (MAX_WAVELENGTH ** (np.arange(0, _EFF, 2, dtype=np.float32) / _EFF)))
        .reshape(1, _N_FREQ))
    return pl.pallas_call(
        _b_body,
        grid=(N_QUERY // blk,),
        in_specs=[
            pl.BlockSpec((blk, 3), lambda i: (i, 0)),
            pl.BlockSpec((HIDDEN, H2), lambda i: (0, 0)),
            pl.BlockSpec((1, H2), lambda i: (0, 0)),
            pl.BlockSpec((1, _N_FREQ), lambda i: (0, 0)),
        ],
        out_specs=pl.BlockSpec((blk, HIDDEN), lambda i: (i, 0)),
        out_shape=jax.ShapeDtypeStruct((N_QUERY, HIDDEN), jnp.uint32),
    )(query_pos, w1b, b_m1, omega)


def _sc_gather_body(a_hbm, b_hbm, gi_hbm, qi_hbm, ha_hbm, hb_hbm,
                    gi_all, qi_all, ar0, br0, ar1, br1, sg0, sg1, so0, so1):
    c = lax.axis_index("c")
    s = lax.axis_index("s")
    wid = s * NUM_CORES + c
    base = wid * EDGES_PER_WORKER

    pltpu.sync_copy(gi_hbm.at[pl.ds(base, EDGES_PER_WORKER)], gi_all)
    pltpu.sync_copy(qi_hbm.at[pl.ds(base, EDGES_PER_WORKER)], qi_all)

    def g_refs(ch, ar, br):
        gi = gi_all.at[pl.ds(ch * CHUNK, CHUNK)]
        qi = qi_all.at[pl.ds(ch * CHUNK, CHUNK)]
        return (a_hbm.at[gi], ar), (b_hbm.at[qi], br)

    def start_gather(ch, ar, br, sg):
        (sa, da), (sb, db) = g_refs(ch, ar, br)
        pltpu.async_copy(sa, da, sg)
        pltpu.async_copy(sb, db, sg)

    def wait_gather(ch, ar, br, sg):
        (sa, da), (sb, db) = g_refs(ch, ar, br)
        pltpu.make_async_copy(sa, da, sg).wait()
        pltpu.make_async_copy(sb, db, sg).wait()

    def start_out(ch, ar, br, so):
        off = base + ch * CHUNK
        pltpu.async_copy(ar, ha_hbm.at[pl.ds(off, CHUNK)], so)
        pltpu.async_copy(br, hb_hbm.at[pl.ds(off, CHUNK)], so)

    def wait_out(ch, ar, br, so):
        off = base + ch * CHUNK
        pltpu.make_async_copy(ar, ha_hbm.at[pl.ds(off, CHUNK)], so).wait()
        pltpu.make_async_copy(br, hb_hbm.at[pl.ds(off, CHUNK)], so).wait()

    start_gather(0, ar0, br0, sg0)

    def body(j, _):
        c0 = 2 * j
        c1 = 2 * j + 1

        @pl.when(j > 0)
        def _():
            wait_out(c1 - 2, ar1, br1, so1)

        start_gather(c1, ar1, br1, sg1)
        wait_gather(c0, ar0, br0, sg0)
        start_out(c0, ar0, br0, so0)

        @pl.when(c0 + 2 < NUM_CHUNKS)
        def _():
            wait_out(c0, ar0, br0, so0)
            start_gather(c0 + 2, ar0, br0, sg0)

        wait_gather(c1, ar1, br1, sg1)
        start_out(c1, ar1, br1, so1)
        return 0

    lax.fori_loop(0, NUM_CHUNKS // 2, body, 0, unroll=False)
    last = NUM_CHUNKS - 1
    wait_gather(last, ar0, br0, sg0)
    start_out(last, ar0, br0, so0)
    wait_out(last, ar0, br0, so0)
    wait_out(NUM_CHUNKS - 2, ar1, br1, so1)


def _sc_gather(a, b, g_idx, q_idx):
    mesh = plsc.VectorSubcoreMesh(core_axis_name="c", subcore_axis_name="s")
    return pl.kernel(
        _sc_gather_body,
        out_type=(
            jax.ShapeDtypeStruct((N_EDGES, HIDDEN), jnp.uint32),
            jax.ShapeDtypeStruct((N_EDGES, HIDDEN), jnp.uint32),
        ),
        mesh=mesh,
        scratch_types=[
            pltpu.VMEM((EDGES_PER_WORKER,), jnp.int32),
            pltpu.VMEM((EDGES_PER_WORKER,), jnp.int32),
            pltpu.VMEM((CHUNK, HIDDEN), jnp.uint32),
            pltpu.VMEM((CHUNK, HIDDEN), jnp.uint32),
            pltpu.VMEM((CHUNK, HIDDEN), jnp.uint32),
            pltpu.VMEM((CHUNK, HIDDEN), jnp.uint32),
            pltpu.SemaphoreType.DMA,
            pltpu.SemaphoreType.DMA,
            pltpu.SemaphoreType.DMA,
            pltpu.SemaphoreType.DMA,
        ],
    )(a, b, g_idx, q_idx)


def _unpack_pair(p):
    lo = lax.bitcast_convert_type((p & 0xFFFF).astype(jnp.uint16), jnp.bfloat16)
    hi = lax.bitcast_convert_type((p >> 16).astype(jnp.uint16), jnp.bfloat16)
    return lo.astype(F32), hi.astype(F32)


def _mlp_body(ha_ref, hb_ref, w2_ref, b2_ref, v_ref):
    alo, ahi = _unpack_pair(ha_ref[...])
    blo, bhi = _unpack_pair(hb_ref[...])
    h = jnp.concatenate([alo + blo, ahi + bhi], axis=1)
    u = _gelu(h)
    v_ref[...] = _gelu(jnp.dot(u, w2_ref[...], preferred_element_type=F32) + b2_ref[...])


def _make_v(ha, hb, w_m2, b_m2):
    blk = 3200
    return pl.pallas_call(
        _mlp_body,
        grid=(N_EDGES // blk,),
        in_specs=[
            pl.BlockSpec((blk, HIDDEN), lambda i: (i, 0)),
            pl.BlockSpec((blk, HIDDEN), lambda i: (i, 0)),
            pl.BlockSpec((H2, HIDDEN), lambda i: (0, 0)),
            pl.BlockSpec((1, HIDDEN), lambda i: (0, 0)),
        ],
        out_specs=pl.BlockSpec((blk, HIDDEN), lambda i: (i, 0)),
        out_shape=jax.ShapeDtypeStruct((N_EDGES, HIDDEN), F32),
    )(ha, hb, w_m2, b_m2)


NQ_HALF = NQ_PAD // 2
ACC_ROWS = 5248
ROWS_PER_TILE_S = ACC_ROWS // NUM_SUBCORES
DUMP_ROW = NQ_HALF
EDGES_PER_SUBCORE = N_EDGES // NUM_SUBCORES
NUM_CHUNKS_S = EDGES_PER_SUBCORE // CHUNK


def _fill_const(buf, rows, val16):
    def frow(r, _):
        for j in range(HIDDEN // 16):
            buf[r, pl.ds(j * 16, 16)] = val16
        return 0

    lax.fori_loop(0, rows, frow, 0, unroll=False)


def _zero_acc_slice(acc_s, v_v, r0):
    for k in range(4):
        pltpu.sync_copy(v_v, acc_s.at[pl.ds(r0 + k * CHUNK, CHUNK)])
    pltpu.sync_copy(v_v.at[pl.ds(0, 8)], acc_s.at[pl.ds(r0 + 4 * CHUNK, 8)])


def _sc_scatter_body(v_hbm, qi_hbm, sums_hbm, cnts_hbm,
                     qi_all, qi2_all, v0, v1, acc_s, sv0, sv1, ss0, ss1, s2):
    c = lax.axis_index("c")
    s = lax.axis_index("s")
    base = s * EDGES_PER_SUBCORE
    lo = c * NQ_HALF
    r0 = s * ROWS_PER_TILE_S

    pltpu.sync_copy(qi_hbm.at[pl.ds(base, EDGES_PER_SUBCORE)], qi_all)

    def premap(i, _):
        for k in range(CHUNK // 16):
            q = qi_all[pl.ds(i * CHUNK + k * 16, 16)] - lo
            in_rng = (q >= 0) & (q < NQ_HALF)
            qi2_all[i, pl.ds(k * 16, 16)] = jnp.where(in_rng, q, DUMP_ROW)
        return 0

    lax.fori_loop(0, NUM_CHUNKS_S, premap, 0, unroll=False)

    _fill_const(v0, CHUNK, jnp.zeros((16,), F32))
    _zero_acc_slice(acc_s, v0, r0)
    plsc.subcore_barrier()

    def start_vload(ch, vb, sv):
        pltpu.async_copy(v_hbm.at[pl.ds(base + ch * CHUNK, CHUNK)], vb, sv)

    def wait_vload(ch, vb, sv):
        pltpu.make_async_copy(v_hbm.at[pl.ds(base + ch * CHUNK, CHUNK)], vb, sv).wait()

    def start_scatter(ch, vb, ss):
        pltpu.async_copy(vb, acc_s.at[qi2_all.at[ch]], ss, add=True)

    def wait_scatter(ch, vb, ss):
        pltpu.make_async_copy(vb, acc_s.at[qi2_all.at[ch]], ss).wait()

    start_vload(0, v0, sv0)

    def body(j, _):
        c0 = 2 * j
        c1 = 2 * j + 1

        @pl.when(j > 0)
        def _():
            wait_scatter(c1 - 2, v1, ss1)

        start_vload(c1, v1, sv1)
        wait_vload(c0, v0, sv0)
        start_scatter(c0, v0, ss0)

        @pl.when(c0 + 2 < NUM_CHUNKS_S)
        def _():
            wait_scatter(c0, v0, ss0)
            start_vload(c0 + 2, v0, sv0)

        wait_vload(c1, v1, sv1)
        start_scatter(c1, v1, ss1)
        return 0

    lax.fori_loop(0, NUM_CHUNKS_S // 2, body, 0, unroll=False)
    wait_scatter(NUM_CHUNKS_S - 2, v0, ss0)
    wait_scatter(NUM_CHUNKS_S - 1, v1, ss1)
    plsc.subcore_barrier()

    pltpu.sync_copy(acc_s.at[pl.ds(r0, ROWS_PER_TILE_S)],
                    sums_hbm.at[c, pl.ds(r0, ROWS_PER_TILE_S)])
    _fill_const(v0, CHUNK, jnp.zeros((16,), F32))
    _zero_acc_slice(acc_s, v0, r0)
    plsc.subcore_barrier()

    _fill_const(v0, CHUNK, jnp.ones((16,), F32))

    def body2(j, _):
        start_scatter(2 * j, v0, s2)
        start_scatter(2 * j + 1, v0, s2)

        @pl.when(j > 0)
        def _():
            wait_scatter(2 * j - 2, v0, s2)
            wait_scatter(2 * j - 1, v0, s2)

        return 0

    lax.fori_loop(0, NUM_CHUNKS_S // 2, body2, 0, unroll=False)
    wait_scatter(NUM_CHUNKS_S - 2, v0, s2)
    wait_scatter(NUM_CHUNKS_S - 1, v0, s2)
    plsc.subcore_barrier()

    pltpu.sync_copy(acc_s.at[pl.ds(r0, ROWS_PER_TILE_S)],
                    cnts_hbm.at[c, pl.ds(r0, ROWS_PER_TILE_S)])


def _sc_scatter(v, q_idx):
    mesh = plsc.VectorSubcoreMesh(core_axis_name="c", subcore_axis_name="s")
    return pl.kernel(
        _sc_scatter_body,
        out_type=(
            jax.ShapeDtypeStruct((NUM_CORES, ACC_ROWS, HIDDEN), F32),
            jax.ShapeDtypeStruct((NUM_CORES, ACC_ROWS, HIDDEN), F32),
        ),
        mesh=mesh,
        scratch_types=[
            pltpu.VMEM((EDGES_PER_SUBCORE,), jnp.int32),
            pltpu.VMEM((NUM_CHUNKS_S, CHUNK), jnp.int32),
            pltpu.VMEM((CHUNK, HIDDEN), F32),
            pltpu.VMEM((CHUNK, HIDDEN), F32),
            pltpu.VMEM_SHARED((ACC_ROWS, HIDDEN), F32),
            pltpu.SemaphoreType.DMA,
            pltpu.SemaphoreType.DMA,
            pltpu.SemaphoreType.DMA,
            pltpu.SemaphoreType.DMA,
            pltpu.SemaphoreType.DMA,
        ],
    )(v, q_idx)


def _fin_body(s_ref, cnt_ref, w3_ref, b3_ref,
              wp1_ref, bp1_ref, wp2_ref, bp2_ref, o_ref):
    sums = s_ref[...]
    cnt = cnt_ref[:, 0:1]
    mbar = sums / jnp.maximum(cnt, 1.0)
    m = jnp.dot(mbar, w3_ref[...], preferred_element_type=F32) + b3_ref[...]
    m = jnp.where(cnt > 0.0, m, 0.0)
    z = _gelu(jnp.dot(m, wp1_ref[...], preferred_element_type=F32) + bp1_ref[...])
    o_ref[...] = jnp.dot(z, wp2_ref[...], preferred_element_type=F32) + bp2_ref[...]


def _finalize(s, cnt, w_m3, b_m3, w_p1, b_p1, w_p2, b_p2):
    blk = 1024
    n_out = 4
    return pl.pallas_call(
        _fin_body,
        grid=(NQ_PAD // blk,),
        in_specs=[
            pl.BlockSpec((blk, HIDDEN), lambda i: (i, 0)),
            pl.BlockSpec((blk, HIDDEN), lambda i: (i, 0)),
            pl.BlockSpec((HIDDEN, HIDDEN), lambda i: (0, 0)),
            pl.BlockSpec((1, HIDDEN), lambda i: (0, 0)),
            pl.BlockSpec((HIDDEN, HIDDEN), lambda i: (0, 0)),
            pl.BlockSpec((1, HIDDEN), lambda i: (0, 0)),
            pl.BlockSpec((HIDDEN, n_out), lambda i: (0, 0)),
            pl.BlockSpec((1, n_out), lambda i: (0, 0)),
        ],
        out_specs=pl.BlockSpec((blk, n_out), lambda i: (i, 0)),
        out_shape=jax.ShapeDtypeStruct((NQ_PAD, n_out), F32),
    )(s, cnt, w_m3, b_m3, w_p1, b_p1, w_p2, b_p2)


def kernel(x, query_pos, grid_to_query_edges, W_proj, b_proj, W_m1, b_m1,
           W_m2, b_m2, W_m3, b_m3, W_p1, b_p1, W_p2, b_p2):
    xf2 = x.reshape(-1, x.shape[-1]).astype(F32)
    q_idx = grid_to_query_edges[:, 0].astype(jnp.int32)
    g_idx = grid_to_query_edges[:, 1].astype(jnp.int32)

    w1t = W_m1[:HIDDEN]
    w1b = W_m1[HIDDEN:]

    a = _make_a(xf2, W_proj, b_proj.reshape(1, -1), w1t)
    b = _make_b(query_pos, w1b, b_m1.reshape(1, -1))
    ha, hb = _sc_gather(a, b, g_idx, q_idx)
    v = _make_v(ha, hb, W_m2, b_m2.reshape(1, -1))
    sums_p, cnts_p = _sc_scatter(v, q_idx)
    sums = jnp.concatenate([sums_p[0, :NQ_HALF], sums_p[1, :NQ_HALF]], axis=0)
    cnts = jnp.concatenate([cnts_p[0, :NQ_HALF], cnts_p[1, :NQ_HALF]], axis=0)
    out = _finalize(sums, cnts,
                    W_m3, b_m3.reshape(1, -1), W_p1, b_p1.reshape(1, -1),
                    W_p2, b_p2.reshape(1, -1))
    return out[:N_QUERY]
